# Initial kernel scaffold; baseline (speedup 1.0000x reference)
import functools

import jax
import jax.numpy as jnp
from jax import lax
from jax.experimental import pallas as pl
from jax.experimental.pallas import tpu as pltpu
from jax.experimental.pallas import tpu_sc as plsc

N = 10000
E = 320000
D = 128
R = 130
EPS = 1e-5

NC = 2
NS = 16
NW = NC * NS
CB = 128
EW = 10112
E_PAD = EW * NW
NCHUNK = EW // CB
N_PAD = 10240
ROWS_PER_TILE = N_PAD // NS
TRASH = N
SW = 144

_mesh = plsc.VectorSubcoreMesh(core_axis_name="c", subcore_axis_name="s")


def _worker_id():
    return lax.axis_index("c") * NS + lax.axis_index("s")


@functools.partial(
    pl.kernel,
    out_type=[
        jax.ShapeDtypeStruct((E_PAD,), jnp.float32),
        jax.ShapeDtypeStruct((NC, N_PAD, 16), jnp.float32),
    ],
    mesh=_mesh,
    scratch_types=[
        pltpu.VMEM((R, D), jnp.float32),
        pltpu.VMEM((1, CB), jnp.int32),
        pltpu.VMEM((1, CB), jnp.int32),
        pltpu.VMEM((1, CB), jnp.int32),
        pltpu.VMEM((1, CB), jnp.int32),
        pltpu.VMEM((CB, D), jnp.float32),
        pltpu.VMEM((CB, D), jnp.float32),
        pltpu.VMEM((CB, 16), jnp.float32),
        pltpu.VMEM((CB,), jnp.float32),
        pltpu.VMEM_SHARED((N_PAD, 16), jnp.float32),
        pltpu.SemaphoreType.DMA,
    ],
)
def _pass1(ent_hbm, rel_hbm, src_hbm, dstg_hbm, dsts_hbm, relid_hbm,
           score_hbm, stats_hbm,
           rel_v, sidx, dgidx, dsidx, ridx, srows, drows, statrows,
           scorebuf, stats_acc, sem):
    cid = lax.axis_index("c")
    sid = lax.axis_index("s")
    wid = _worker_id()

    pltpu.sync_copy(rel_hbm, rel_v)

    zero16 = jnp.zeros((16,), jnp.float32)
    lane = lax.iota(jnp.int32, 16)
    onehot1 = jnp.where(lane == 1, 1.0, 0.0).astype(jnp.float32)

    def _zrow(i, _):
        statrows[i] = zero16
        return 0
    lax.fori_loop(0, CB, _zrow, 0)
    for k in range(ROWS_PER_TILE // CB):
        pltpu.sync_copy(
            statrows, stats_acc.at[pl.ds(sid * ROWS_PER_TILE + k * CB, CB)])

    def _initrow(i, _):
        statrows[i] = onehot1
        return 0
    lax.fori_loop(0, CB, _initrow, 0)
    plsc.subcore_barrier()

    def _chunk(c, _):
        base = wid * EW + c * CB
        pltpu.sync_copy(src_hbm.at[pl.ds(base, CB)], sidx.at[0])
        pltpu.sync_copy(dstg_hbm.at[pl.ds(base, CB)], dgidx.at[0])
        pltpu.sync_copy(dsts_hbm.at[pl.ds(base, CB)], dsidx.at[0])
        pltpu.sync_copy(relid_hbm.at[pl.ds(base, CB)], ridx.at[0])
        cp1 = pltpu.async_copy(ent_hbm.at[sidx.at[0]], srows, sem)
        cp2 = pltpu.async_copy(ent_hbm.at[dgidx.at[0]], drows, sem)
        cp1.wait()
        cp2.wait()

        def _edge(e, _):
            rid = ridx[0, e]
            acc = jnp.zeros((16,), jnp.float32)
            for j in range(D // 16):
                sl = pl.ds(j * 16, 16)
                acc = acc + srows[e, sl] * drows[e, sl] * rel_v[rid, sl]
            s = jnp.sum(acc)
            scorebuf[e] = s
            statrows[e, 0] = s
            return 0
        lax.fori_loop(0, CB, _edge, 0)

        pltpu.sync_copy(scorebuf, score_hbm.at[pl.ds(base, CB)])
        pltpu.sync_copy(statrows, stats_acc.at[dsidx.at[0]], add=True)
        return 0
    lax.fori_loop(0, NCHUNK, _chunk, 0)

    plsc.subcore_barrier()
    row0 = sid * ROWS_PER_TILE
    pltpu.sync_copy(stats_acc.at[pl.ds(row0, ROWS_PER_TILE)],
                    stats_hbm.at[cid].at[pl.ds(row0, ROWS_PER_TILE)])


def _mean_body(stats_ref, m_ref):
    s = stats_ref[0] + stats_ref[1]
    m = s[:, 0:1] / jnp.maximum(s[:, 1:2], 1.0)
    m_ref[...] = jnp.reshape(m, (N_PAD // 128, 128))


def _seg_mean(stats):
    return pl.pallas_call(
        _mean_body,
        out_shape=jax.ShapeDtypeStruct((N_PAD // 128, 128), jnp.float32),
    )(stats)


@functools.partial(
    pl.kernel,
    out_type=jax.ShapeDtypeStruct((NC, N_PAD, SW), jnp.float32),
    mesh=_mesh,
    scratch_types=[
        pltpu.VMEM((R, D), jnp.float32),
        pltpu.VMEM((N_PAD,), jnp.float32),
        pltpu.VMEM((1, CB), jnp.int32),
        pltpu.VMEM((1, CB), jnp.int32),
        pltpu.VMEM((1, CB), jnp.int32),
        pltpu.VMEM((CB, D), jnp.float32),
        pltpu.VMEM((CB,), jnp.float32),
        pltpu.VMEM((CB,), jnp.float32),
        pltpu.VMEM((CB, SW), jnp.float32),
        pltpu.VMEM_SHARED((N_PAD, SW), jnp.float32),
        pltpu.SemaphoreType.DMA,
    ],
)
def _pass2(ent_hbm, rel_hbm, src_hbm, dsts_hbm, relid_hbm, score_hbm, m_hbm,
           acc_hbm,
           rel_v, m_v, sidx, dsidx, ridx, srows, scorebuf, exbuf, scatbuf,
           acc, sem):
    cid = lax.axis_index("c")
    sid = lax.axis_index("s")
    wid = _worker_id()

    pltpu.sync_copy(rel_hbm, rel_v)
    pltpu.sync_copy(m_hbm, m_v)

    zero16 = jnp.zeros((16,), jnp.float32)

    def _zrow(i, _):
        for j in range(SW // 16):
            scatbuf[i, pl.ds(j * 16, 16)] = zero16
        return 0
    lax.fori_loop(0, CB, _zrow, 0)
    for k in range(ROWS_PER_TILE // CB):
        pltpu.sync_copy(scatbuf,
                        acc.at[pl.ds(sid * ROWS_PER_TILE + k * CB, CB)])
    plsc.subcore_barrier()

    def _chunk(c, _):
        base = wid * EW + c * CB
        pltpu.sync_copy(src_hbm.at[pl.ds(base, CB)], sidx.at[0])
        pltpu.sync_copy(dsts_hbm.at[pl.ds(base, CB)], dsidx.at[0])
        pltpu.sync_copy(relid_hbm.at[pl.ds(base, CB)], ridx.at[0])
        pltpu.sync_copy(score_hbm.at[pl.ds(base, CB)], scorebuf)
        pltpu.async_copy(ent_hbm.at[sidx.at[0]], srows, sem).wait()

        for v in range(CB // 16):
            sl = pl.ds(v * 16, 16)
            dstvec = dsidx[0, sl]
            mvec = plsc.load_gather(m_v, [dstvec])
            exbuf[sl] = jnp.exp(scorebuf[sl] - mvec)

        def _edge(e, _):
            rid = ridx[0, e]
            ex = exbuf[e]
            for j in range(D // 16):
                sl = pl.ds(j * 16, 16)
                scatbuf[e, sl] = srows[e, sl] * rel_v[rid, sl] * ex
            scatbuf[e, D] = ex
            return 0
        lax.fori_loop(0, CB, _edge, 0)

        pltpu.sync_copy(scatbuf, acc.at[dsidx.at[0]], add=True)
        return 0
    lax.fori_loop(0, NCHUNK, _chunk, 0)

    plsc.subcore_barrier()
    row0 = sid * ROWS_PER_TILE
    pltpu.sync_copy(acc.at[pl.ds(row0, ROWS_PER_TILE)],
                    acc_hbm.at[cid].at[pl.ds(row0, ROWS_PER_TILE)])


def _final_body(acc_ref, w_ref, g_ref, b_ref, out_ref):
    a = acc_ref[0] + acc_ref[1]
    num = a[0:N, 0:D]
    den = a[0:N, D:D + 1]
    neigh = num / jnp.maximum(den, 1e-30)
    out = jnp.dot(neigh, w_ref[...], preferred_element_type=jnp.float32)
    mean = jnp.mean(out, axis=0, keepdims=True)
    var = jnp.mean((out - mean) ** 2, axis=0, keepdims=True)
    out = (out - mean) / jnp.sqrt(var + EPS) * g_ref[...] + b_ref[...]
    out_ref[...] = jnp.tanh(out)


def _final(acc, neigh_w, bn_gamma, bn_beta):
    return pl.pallas_call(
        _final_body,
        out_shape=jax.ShapeDtypeStruct((N, D), jnp.float32),
    )(acc, neigh_w, bn_gamma.reshape(1, D), bn_beta.reshape(1, D))


def kernel(ent_emb, rel_emb, edge_index, rel_id, neigh_w, bn_gamma, bn_beta):
    src = edge_index[0]
    dst = edge_index[1]
    pad = E_PAD - E
    zpad = jnp.zeros((pad,), jnp.int32)
    src_p = jnp.concatenate([src, zpad])
    dstg_p = jnp.concatenate([dst, zpad])
    dsts_p = jnp.concatenate([dst, jnp.full((pad,), TRASH, jnp.int32)])
    rel_p = jnp.concatenate([rel_id, zpad])

    score, stats = _pass1(ent_emb, rel_emb, src_p, dstg_p, dsts_p, rel_p)
    m = _seg_mean(stats).reshape(N_PAD)
    acc = _pass2(ent_emb, rel_emb, src_p, dsts_p, rel_p, score, m)
    return _final(acc, neigh_w, bn_gamma, bn_beta)


# speedup vs baseline: 4.9800x; 4.9800x over previous
import functools

import jax
import jax.numpy as jnp
from jax import lax
from jax.experimental import pallas as pl
from jax.experimental.pallas import tpu as pltpu
from jax.experimental.pallas import tpu_sc as plsc

N = 10000
E = 320000
D = 128
R = 130
EPS = 1e-5

NC = 2
NS = 16
NW = NC * NS
CB = 128
EW = 10112
E_PAD = EW * NW
NCHUNK = EW // CB
CB2 = 64
N_PAD = 10240
ROWS_PER_TILE = N_PAD // NS
TRASH = N


_mesh = plsc.VectorSubcoreMesh(core_axis_name="c", subcore_axis_name="s")


def _worker_id():
    return lax.axis_index("c") * NS + lax.axis_index("s")


@functools.partial(
    pl.kernel,
    out_type=[
        jax.ShapeDtypeStruct((E_PAD,), jnp.float32),
        jax.ShapeDtypeStruct((E_PAD, D), jnp.float32),
        jax.ShapeDtypeStruct((NC, N_PAD, 16), jnp.float32),
    ],
    mesh=_mesh,
    compiler_params=pltpu.CompilerParams(needs_layout_passes=False,
                                         use_tc_tiling_on_sc=False),
    scratch_types=[
        pltpu.VMEM((R, D), jnp.float32),
        pltpu.VMEM((CB,), jnp.int32),
        pltpu.VMEM((CB,), jnp.int32),
        pltpu.VMEM((1, CB), jnp.int32),
        pltpu.VMEM((1, CB), jnp.int32),
        pltpu.VMEM((CB, D), jnp.float32),
        pltpu.VMEM((CB, D), jnp.float32),
        pltpu.VMEM((CB, 16), jnp.float32),
        pltpu.VMEM((CB,), jnp.float32),
        pltpu.VMEM_SHARED((N_PAD, 16), jnp.float32),
        pltpu.SemaphoreType.DMA,
    ],
)
def _pass1(ent_hbm, rel_hbm, src_hbm, dstg_hbm, dsts_hbm, relid_hbm,
           score_hbm, comp_hbm, stats_hbm,
           rel_v, sidx, dgidx, dsidx, ridx, srows, drows, statrows,
           scorebuf, stats_acc, sem):
    cid = lax.axis_index("c")
    sid = lax.axis_index("s")
    wid = _worker_id()
    lane = lax.iota(jnp.int32, 16)
    zero16 = jnp.zeros((16,), jnp.float32)
    onehot1 = jnp.where(lane == 1, 1.0, 0.0).astype(jnp.float32)

    pltpu.sync_copy(rel_hbm, rel_v)

    def _zrow(i, _):
        statrows[i] = zero16
        return 0
    lax.fori_loop(0, CB, _zrow, 0)
    for k in range(ROWS_PER_TILE // CB):
        pltpu.sync_copy(
            statrows, stats_acc.at[pl.ds(sid * ROWS_PER_TILE + k * CB, CB)])
    plsc.subcore_barrier()

    def _chunk(c, _):
        base = wid * EW + c * CB
        pltpu.sync_copy(src_hbm.at[pl.ds(base, CB)], sidx)
        pltpu.sync_copy(dstg_hbm.at[pl.ds(base, CB)], dgidx)
        pltpu.sync_copy(dsts_hbm.at[pl.ds(base, CB)], dsidx.at[0])
        pltpu.sync_copy(relid_hbm.at[pl.ds(base, CB)], ridx.at[0])
        cp1 = pltpu.async_copy(ent_hbm.at[sidx], srows, sem)
        cp2 = pltpu.async_copy(ent_hbm.at[dgidx], drows, sem)
        cp1.wait()
        cp2.wait()

        def _group(v, _):
            sl = pl.ds(v * 16, 16)
            rid_vec = ridx[0, sl]
            svec = zero16
            for l in range(16):
                e = v * 16 + l
                rid = rid_vec[l]
                acc = zero16
                for j in range(D // 16):
                    slj = pl.ds(j * 16, 16)
                    c = srows[e, slj] * rel_v[rid, slj]
                    acc = acc + c * drows[e, slj]
                    srows[e, slj] = c
                s = jnp.sum(acc)
                svec = jnp.where(lane == l, s, svec)
                statrows[e] = jnp.where(lane == 0, s, onehot1)
            scorebuf[sl] = svec
            return 0
        lax.fori_loop(0, CB // 16, _group, 0)

        pltpu.sync_copy(scorebuf, score_hbm.at[pl.ds(base, CB)])
        pltpu.sync_copy(srows, comp_hbm.at[pl.ds(base, CB)])
        pltpu.sync_copy(statrows, stats_acc.at[dsidx.at[0]], add=True)
        return 0
    lax.fori_loop(0, NCHUNK, _chunk, 0)

    plsc.subcore_barrier()
    row0 = sid * ROWS_PER_TILE
    pltpu.sync_copy(stats_acc.at[pl.ds(row0, ROWS_PER_TILE)],
                    stats_hbm.at[cid].at[pl.ds(row0, ROWS_PER_TILE)])


def _mean_body(stats_ref, m_ref):
    s = stats_ref[0] + stats_ref[1]
    m = s[:, 0:1] / jnp.maximum(s[:, 1:2], 1.0)
    m_ref[...] = jnp.reshape(m, (N_PAD // 128, 128))


def _seg_mean(stats):
    return pl.pallas_call(
        _mean_body,
        out_shape=jax.ShapeDtypeStruct((N_PAD // 128, 128), jnp.float32),
    )(stats)


@functools.partial(
    pl.kernel,
    out_type=[
        jax.ShapeDtypeStruct((NC, N_PAD, D), jnp.float32),
        jax.ShapeDtypeStruct((NC, N_PAD, 16), jnp.float32),
    ],
    mesh=_mesh,
    compiler_params=pltpu.CompilerParams(needs_layout_passes=False,
                                         use_tc_tiling_on_sc=False),
    scratch_types=[
        pltpu.VMEM((1, CB2), jnp.int32),
        pltpu.VMEM((N_PAD,), jnp.float32),
        pltpu.VMEM((CB2,), jnp.float32),
        pltpu.VMEM((CB2, D), jnp.float32),
        pltpu.VMEM((CB2, 16), jnp.float32),
        pltpu.VMEM_SHARED((N_PAD, D), jnp.float32),
        pltpu.VMEM_SHARED((N_PAD, 16), jnp.float32),
        pltpu.SemaphoreType.DMA,
    ],
)
def _pass2(comp_hbm, dsts_hbm, score_hbm, m_hbm,
           accn_hbm, accd_hbm,
           dsidx, m_v, scorebuf, scatbuf, exrows,
           acc, accd, sem):
    cid = lax.axis_index("c")
    sid = lax.axis_index("s")
    wid = _worker_id()
    lane = lax.iota(jnp.int32, 16)
    zero16 = jnp.zeros((16,), jnp.float32)

    pltpu.sync_copy(m_hbm, m_v)

    def _zrow(i, _):
        for j in range(D // 16):
            scatbuf[i, pl.ds(j * 16, 16)] = zero16
        exrows[i] = zero16
        return 0
    lax.fori_loop(0, CB2, _zrow, 0)
    for k in range(ROWS_PER_TILE // CB2):
        pltpu.sync_copy(scatbuf,
                        acc.at[pl.ds(sid * ROWS_PER_TILE + k * CB2, CB2)])
        pltpu.sync_copy(exrows,
                        accd.at[pl.ds(sid * ROWS_PER_TILE + k * CB2, CB2)])
    plsc.subcore_barrier()

    def _chunk(c, _):
        base = wid * EW + c * CB2
        pltpu.sync_copy(dsts_hbm.at[pl.ds(base, CB2)], dsidx.at[0])
        pltpu.sync_copy(score_hbm.at[pl.ds(base, CB2)], scorebuf)
        pltpu.sync_copy(comp_hbm.at[pl.ds(base, CB2)], scatbuf)

        def _group(v, _):
            sl = pl.ds(v * 16, 16)
            dstvec = dsidx[0, sl]
            mvec = plsc.load_gather(m_v, [dstvec])
            ex = jnp.exp(scorebuf[sl] - mvec)
            for l in range(16):
                e = v * 16 + l
                exl = ex[l]
                for j in range(D // 16):
                    slj = pl.ds(j * 16, 16)
                    scatbuf[e, slj] = scatbuf[e, slj] * exl
                exrows[e] = jnp.where(lane == 0, exl, 0.0)
            return 0
        lax.fori_loop(0, CB2 // 16, _group, 0)

        pltpu.sync_copy(scatbuf, acc.at[dsidx.at[0]], add=True)
        pltpu.sync_copy(exrows, accd.at[dsidx.at[0]], add=True)
        return 0
    lax.fori_loop(0, EW // CB2, _chunk, 0)

    plsc.subcore_barrier()
    row0 = sid * ROWS_PER_TILE
    pltpu.sync_copy(acc.at[pl.ds(row0, ROWS_PER_TILE)],
                    accn_hbm.at[cid].at[pl.ds(row0, ROWS_PER_TILE)])
    pltpu.sync_copy(accd.at[pl.ds(row0, ROWS_PER_TILE)],
                    accd_hbm.at[cid].at[pl.ds(row0, ROWS_PER_TILE)])


def _final_body(accn_ref, accd_ref, w_ref, g_ref, b_ref, out_ref):
    num = (accn_ref[0] + accn_ref[1])[0:N]
    den = (accd_ref[0] + accd_ref[1])[0:N, 0:1]
    neigh = num / jnp.maximum(den, 1e-30)
    out = jnp.dot(neigh, w_ref[...], preferred_element_type=jnp.float32)
    mean = jnp.mean(out, axis=0, keepdims=True)
    var = jnp.mean((out - mean) ** 2, axis=0, keepdims=True)
    out = (out - mean) / jnp.sqrt(var + EPS) * g_ref[...] + b_ref[...]
    out_ref[...] = jnp.tanh(out)


def _final(accn, accd, neigh_w, bn_gamma, bn_beta):
    return pl.pallas_call(
        _final_body,
        out_shape=jax.ShapeDtypeStruct((N, D), jnp.float32),
    )(accn, accd, neigh_w, bn_gamma.reshape(1, D), bn_beta.reshape(1, D))


def kernel(ent_emb, rel_emb, edge_index, rel_id, neigh_w, bn_gamma, bn_beta):
    src = edge_index[0]
    dst = edge_index[1]
    pad = E_PAD - E
    zpad = jnp.zeros((pad,), jnp.int32)
    src_p = jnp.concatenate([src, zpad])
    dstg_p = jnp.concatenate([dst, zpad])
    dsts_p = jnp.concatenate([dst, jnp.full((pad,), TRASH, jnp.int32)])
    rel_p = jnp.concatenate([rel_id, zpad])

    score, comp, stats = _pass1(ent_emb, rel_emb, src_p, dstg_p, dsts_p,
                                rel_p)
    m = _seg_mean(stats).reshape(N_PAD)
    accn, accd = _pass2(comp, dsts_p, score, m)
    return _final(accn, accd, neigh_w, bn_gamma, bn_beta)
